# baseline (speedup 1.0000x reference)
import functools

import numpy as np

import jax
import jax.numpy as jnp
from jax import lax
from jax.experimental import pallas as pl
from jax.experimental.pallas import tpu as pltpu
from jax.experimental.pallas import tpu_sc as plsc

N = 10000
E = 320000
NC = 2
NS = 16
NW = NC * NS
EW = E // NW
C = 80
NCHUNK = EW // C
N_PAD = 10240
ROWS_PER_SUB = N_PAD // NS


NE = 8
NMAIN = 120
NTAIL = NCHUNK - NMAIN


def _make_spmm(d, nr, wgap, packed=False):
  mesh = plsc.VectorSubcoreMesh(core_axis_name="c", subcore_axis_name="s")

  @functools.partial(
      pl.kernel,
      out_type=jax.ShapeDtypeStruct((NC, N_PAD, d), jnp.float32),
      mesh=mesh,
      compiler_params=pltpu.CompilerParams(use_tc_tiling_on_sc=False,
                                           needs_layout_passes=False),
      scratch_types=(
          [pltpu.VMEM((2, C), jnp.int32) for _ in range(NE)]
          + [pltpu.VMEM((C,), jnp.float32) for _ in range(NE)]
          + [pltpu.VMEM((C, d), jnp.float32) for _ in range(nr)]
          + ([pltpu.VMEM((C, d // 2), jnp.int32) for _ in range(nr)]
             if packed else [])
          + [pltpu.VMEM_SHARED((N_PAD, d), jnp.float32)]
          + [pltpu.SemaphoreType.DMA for _ in range(2 * NE + nr)]
      ),
  )
  def spmm(x_hbm, ei_hbm, vals_hbm, zeros_hbm, out_hbm, *rest):
    ebuf = rest[:NE]
    valb = rest[NE:2 * NE]
    rows = rest[2 * NE:2 * NE + nr]
    ng = 2 * nr if packed else nr
    gbuf = rest[2 * NE + nr:2 * NE + ng] if packed else rows
    accum = rest[2 * NE + ng]
    esem = rest[2 * NE + ng + 1:2 * NE + ng + 1 + NE]
    ssem = rest[2 * NE + ng + 1 + NE:2 * NE + ng + 1 + 2 * NE]
    gsem = rest[2 * NE + ng + 1 + 2 * NE:]
    cid = lax.axis_index("c")
    sid = lax.axis_index("s")
    wid = cid * NS + sid
    ebase = wid * EW

    def load_idx(chunk, be):
      off = ebase + chunk * C
      pltpu.async_copy(ei_hbm.at[:, pl.ds(off, C)], ebuf[be], esem[be])
      pltpu.async_copy(vals_hbm.at[pl.ds(off, C)], valb[be], esem[be])

    def wait_idx(be):
      pltpu.make_async_copy(ei_hbm.at[:, pl.ds(0, C)], ebuf[be],
                            esem[be]).wait()
      pltpu.make_async_copy(vals_hbm.at[pl.ds(0, C)], valb[be],
                            esem[be]).wait()

    def gather(be, br):
      pltpu.async_copy(x_hbm.at[ebuf[be].at[1]], gbuf[br], gsem[br])

    def wait_gather(be, br):
      pltpu.make_async_copy(x_hbm.at[ebuf[be].at[1]], gbuf[br],
                            gsem[br]).wait()

    def scatter(be, br):
      pltpu.async_copy(rows[br], accum.at[ebuf[be].at[0]], ssem[be], add=True)

    def wait_scatter(be, br):
      pltpu.make_async_copy(rows[br], accum.at[ebuf[be].at[0]],
                            ssem[be]).wait()

    def scale(be, br):
      def group_body(g, c2):
        val16 = valb[be][pl.ds(g * 16, 16)]
        for l in range(16):
          e = g * 16 + l
          v = val16[l]
          if packed:
            for j in range(d // 32):
              v32 = gbuf[br][e, pl.ds(j * 16, 16)]
              lo = plsc.bitcast(v32 << 16, jnp.float32)
              hi = plsc.bitcast(v32 & jnp.int32(-65536), jnp.float32)
              rows[br][e, pl.ds(j * 32, 16)] = lo * v
              rows[br][e, pl.ds(j * 32 + 16, 16)] = hi * v
          else:
            for j in range(d // 16):
              sl = pl.ds(j * 16, 16)
              rows[br][e, sl] = rows[br][e, sl] * v
        return c2
      lax.fori_loop(0, C // 16, group_body, 0)

    for c in range(3):
      load_idx(c, c)
    for c in range(2):
      wait_idx(c)
      gather(c, c)

    r0 = sid * ROWS_PER_SUB
    pltpu.sync_copy(zeros_hbm.at[pl.ds(r0, ROWS_PER_SUB), :],
                    accum.at[pl.ds(r0, ROWS_PER_SUB), :])
    plsc.subcore_barrier()

    def step(k, k8, u):
      if u < wgap:
        @pl.when(k8 >= 1)
        def _wait_prev_scatter():
          wait_scatter((u - wgap) % NE, (u - wgap) % nr)
      else:
        wait_scatter((u - wgap) % NE, (u - wgap) % nr)
      load_idx(k + 3, (u + 3) % NE)
      wait_idx((u + 2) % NE)
      gather((u + 2) % NE, (u + 2) % nr)
      wait_gather(u % NE, u % nr)
      scale(u % NE, u % nr)
      scatter(u % NE, u % nr)

    def outer_body(k8, carry):
      for u in range(NE):
        step(k8 * NE + u, k8, u)
      return carry

    lax.fori_loop(0, NMAIN // NE, outer_body, 0)

    for k in range(NMAIN, NCHUNK):
      wait_scatter((k - wgap) % NE, (k - wgap) % nr)
      if k + 3 < NCHUNK:
        load_idx(k + 3, (k + 3) % NE)
      if k + 2 < NCHUNK:
        wait_idx((k + 2) % NE)
        gather((k + 2) % NE, (k + 2) % nr)
      wait_gather(k % NE, k % nr)
      scale(k % NE, k % nr)
      scatter(k % NE, k % nr)
    for k in range(NCHUNK - wgap, NCHUNK):
      wait_scatter(k % NE, k % nr)
    plsc.subcore_barrier()

    pltpu.sync_copy(accum.at[pl.ds(r0, ROWS_PER_SUB), :],
                    out_hbm.at[cid, pl.ds(r0, ROWS_PER_SUB), :])

  return spmm


_spmm128 = _make_spmm(128, nr=4, wgap=2)
_spmm64 = _make_spmm(64, nr=8, wgap=4, packed=True)


def _half_perm(d):
  p = []
  for q in range(d):
    j, r = divmod(q, 32)
    p.append(32 * j + (r // 2 if r % 2 == 0 else 16 + r // 2))
  return np.array(p, dtype=np.int32)


_PERM64 = _half_perm(64)

_R = 1000


def _dense1_body(p0, p1, w1, b1, w2, out):
  y = p0[0] + p1[0]
  h = jnp.maximum(
      lax.dot_general(y, w1[...], (((1,), (0,)), ((), ())),
                      preferred_element_type=jnp.float32) + b1[...], 0.0)
  out[...] = lax.dot_general(h, w2[...], (((1,), (0,)), ((), ())),
                             preferred_element_type=jnp.float32)


def _dense1(p, w1, b1, w2):
  grid = (N // _R,)
  return pl.pallas_call(
      _dense1_body,
      grid=grid,
      in_specs=[
          pl.BlockSpec((1, _R, 128), lambda i: (0, i, 0)),
          pl.BlockSpec((1, _R, 128), lambda i: (1, i, 0)),
          pl.BlockSpec((128, 128), lambda i: (0, 0)),
          pl.BlockSpec((1, 128), lambda i: (0, 0)),
          pl.BlockSpec((128, 64), lambda i: (0, 0)),
      ],
      out_specs=pl.BlockSpec((_R, 64), lambda i: (i, 0)),
      out_shape=jax.ShapeDtypeStruct((N, 64), jnp.float32),
  )(p, p, w1, b1, w2)


def _dense2_body(q0, q1, b2, out):
  y = q0[0] + q1[0] + b2[...]
  m = jnp.max(y, axis=1, keepdims=True)
  s = y - m
  out[...] = s - jnp.log(jnp.sum(jnp.exp(s), axis=1, keepdims=True))


def _dense2(q, b2):
  grid = (N // _R,)
  return pl.pallas_call(
      _dense2_body,
      grid=grid,
      in_specs=[
          pl.BlockSpec((1, _R, 64), lambda i: (0, i, 0)),
          pl.BlockSpec((1, _R, 64), lambda i: (1, i, 0)),
          pl.BlockSpec((1, 64), lambda i: (0, 0)),
      ],
      out_specs=pl.BlockSpec((_R, 64), lambda i: (i, 0)),
      out_shape=jax.ShapeDtypeStruct((N, 64), jnp.float32),
  )(q, q, b2)


def kernel(input, edge_index, adj_values, W1, b1, W2, b2):
  z128 = jnp.zeros((N_PAD, 128), jnp.float32)
  z64 = jnp.zeros((N_PAD, 64), jnp.float32)

  p = _spmm128(input, edge_index, adj_values, z128)
  z = _dense1(p, W1, b1.reshape(1, 128), W2[:, _PERM64])
  zpk = lax.bitcast_convert_type(
      z.astype(jnp.bfloat16).reshape(N, 32, 2), jnp.int32)
  q = _spmm64(zpk, edge_index, adj_values, z64)
  return _dense2(q, b2.reshape(1, 64))


# speedup vs baseline: 1.2864x; 1.2864x over previous
import functools

import numpy as np

import jax
import jax.numpy as jnp
from jax import lax
from jax.experimental import pallas as pl
from jax.experimental.pallas import tpu as pltpu
from jax.experimental.pallas import tpu_sc as plsc

N = 10000
E = 320000
NC = 2
NS = 16
NW = NC * NS
EW = E // NW
C = 80
NCHUNK = EW // C
N_PAD = 10240
ROWS_PER_SUB = N_PAD // NS


NE = 8
NMAIN = 120
NTAIL = NCHUNK - NMAIN


def _make_spmm(d, nr, wgap, packed=False):
  mesh = plsc.VectorSubcoreMesh(core_axis_name="c", subcore_axis_name="s")

  @functools.partial(
      pl.kernel,
      out_type=jax.ShapeDtypeStruct((NC, N_PAD, d), jnp.float32),
      mesh=mesh,
      compiler_params=pltpu.CompilerParams(use_tc_tiling_on_sc=False,
                                           needs_layout_passes=False),
      scratch_types=(
          [pltpu.VMEM((2, C), jnp.int32) for _ in range(NE)]
          + [pltpu.VMEM((C,), jnp.float32) for _ in range(NE)]
          + [pltpu.VMEM((C, d), jnp.float32) for _ in range(nr)]
          + ([pltpu.VMEM((C, d // 2), jnp.int32) for _ in range(nr)]
             if packed else [])
          + [pltpu.VMEM_SHARED((N_PAD, d), jnp.float32)]
          + [pltpu.SemaphoreType.DMA for _ in range(2 * NE + nr)]
      ),
  )
  def spmm(x_hbm, ei_hbm, vals_hbm, zeros_hbm, out_hbm, *rest):
    ebuf = rest[:NE]
    valb = rest[NE:2 * NE]
    rows = rest[2 * NE:2 * NE + nr]
    ng = 2 * nr if packed else nr
    gbuf = rest[2 * NE + nr:2 * NE + ng] if packed else rows
    accum = rest[2 * NE + ng]
    esem = rest[2 * NE + ng + 1:2 * NE + ng + 1 + NE]
    ssem = rest[2 * NE + ng + 1 + NE:2 * NE + ng + 1 + 2 * NE]
    gsem = rest[2 * NE + ng + 1 + 2 * NE:]
    cid = lax.axis_index("c")
    sid = lax.axis_index("s")
    wid = cid * NS + sid
    ebase = wid * EW

    def load_idx(chunk, be):
      off = ebase + chunk * C
      pltpu.async_copy(ei_hbm.at[:, pl.ds(off, C)], ebuf[be], esem[be])
      pltpu.async_copy(vals_hbm.at[pl.ds(off, C)], valb[be], esem[be])

    def wait_idx(be):
      pltpu.make_async_copy(ei_hbm.at[:, pl.ds(0, C)], ebuf[be],
                            esem[be]).wait()
      pltpu.make_async_copy(vals_hbm.at[pl.ds(0, C)], valb[be],
                            esem[be]).wait()

    def gather(be, br):
      pltpu.async_copy(x_hbm.at[ebuf[be].at[1]], gbuf[br], gsem[br])

    def wait_gather(be, br):
      pltpu.make_async_copy(x_hbm.at[ebuf[be].at[1]], gbuf[br],
                            gsem[br]).wait()

    def scatter(be, br):
      pltpu.async_copy(rows[br], accum.at[ebuf[be].at[0]], ssem[be], add=True)

    def wait_scatter(be, br):
      pltpu.make_async_copy(rows[br], accum.at[ebuf[be].at[0]],
                            ssem[be]).wait()

    def scale(be, br):
      def group_body(g, c2):
        val16 = valb[be][pl.ds(g * 16, 16)]
        for l in range(16):
          e = g * 16 + l
          v = val16[l]
          if packed:
            for j in range(d // 32):
              v32 = gbuf[br][e, pl.ds(j * 16, 16)]
              lo = plsc.bitcast(v32 << 16, jnp.float32)
              hi = plsc.bitcast(v32 & jnp.int32(-65536), jnp.float32)
              rows[br][e, pl.ds(j * 32, 16)] = lo * v
              rows[br][e, pl.ds(j * 32 + 16, 16)] = hi * v
          else:
            for j in range(d // 16):
              sl = pl.ds(j * 16, 16)
              rows[br][e, sl] = rows[br][e, sl] * v
        return c2
      lax.fori_loop(0, C // 16, group_body, 0)

    for c in range(3):
      load_idx(c, c)
    for c in range(2):
      wait_idx(c)
      gather(c, c)

    r0 = sid * ROWS_PER_SUB
    pltpu.sync_copy(zeros_hbm.at[pl.ds(r0, ROWS_PER_SUB), :],
                    accum.at[pl.ds(r0, ROWS_PER_SUB), :])
    plsc.subcore_barrier()

    def step(k, k8, u):
      if u < wgap:
        @pl.when(k8 >= 1)
        def _wait_prev_scatter():
          wait_scatter((u - wgap) % NE, (u - wgap) % nr)
      else:
        wait_scatter((u - wgap) % NE, (u - wgap) % nr)
      load_idx(k + 3, (u + 3) % NE)
      wait_idx((u + 2) % NE)
      gather((u + 2) % NE, (u + 2) % nr)
      wait_gather(u % NE, u % nr)
      scale(u % NE, u % nr)
      scatter(u % NE, u % nr)

    def outer_body(k8, carry):
      for u in range(NE):
        step(k8 * NE + u, k8, u)
      return carry

    lax.fori_loop(0, NMAIN // NE, outer_body, 0)

    for k in range(NMAIN, NCHUNK):
      wait_scatter((k - wgap) % NE, (k - wgap) % nr)
      if k + 3 < NCHUNK:
        load_idx(k + 3, (k + 3) % NE)
      if k + 2 < NCHUNK:
        wait_idx((k + 2) % NE)
        gather((k + 2) % NE, (k + 2) % nr)
      wait_gather(k % NE, k % nr)
      scale(k % NE, k % nr)
      scatter(k % NE, k % nr)
    for k in range(NCHUNK - wgap, NCHUNK):
      wait_scatter(k % NE, k % nr)
    plsc.subcore_barrier()

    pltpu.sync_copy(accum.at[pl.ds(r0, ROWS_PER_SUB), :],
                    out_hbm.at[cid, pl.ds(r0, ROWS_PER_SUB), :])

  return spmm


_spmm128 = _make_spmm(128, nr=4, wgap=2)
_spmm64 = _make_spmm(64, nr=8, wgap=4)

_R = 1000


def _dense1_body(p0, p1, w1, b1, w2, out):
  y = p0[0] + p1[0]
  h = jnp.maximum(
      lax.dot_general(y, w1[...], (((1,), (0,)), ((), ())),
                      preferred_element_type=jnp.float32) + b1[...], 0.0)
  out[...] = lax.dot_general(h, w2[...], (((1,), (0,)), ((), ())),
                             preferred_element_type=jnp.float32)


def _dense1(p, w1, b1, w2):
  grid = (N // _R,)
  return pl.pallas_call(
      _dense1_body,
      grid=grid,
      in_specs=[
          pl.BlockSpec((1, _R, 128), lambda i: (0, i, 0)),
          pl.BlockSpec((1, _R, 128), lambda i: (1, i, 0)),
          pl.BlockSpec((128, 128), lambda i: (0, 0)),
          pl.BlockSpec((1, 128), lambda i: (0, 0)),
          pl.BlockSpec((128, 64), lambda i: (0, 0)),
      ],
      out_specs=pl.BlockSpec((_R, 64), lambda i: (i, 0)),
      out_shape=jax.ShapeDtypeStruct((N, 64), jnp.float32),
  )(p, p, w1, b1, w2)


def _dense2_body(q0, q1, b2, out):
  y = q0[0] + q1[0] + b2[...]
  m = jnp.max(y, axis=1, keepdims=True)
  s = y - m
  out[...] = s - jnp.log(jnp.sum(jnp.exp(s), axis=1, keepdims=True))


def _dense2(q, b2):
  grid = (N // _R,)
  return pl.pallas_call(
      _dense2_body,
      grid=grid,
      in_specs=[
          pl.BlockSpec((1, _R, 64), lambda i: (0, i, 0)),
          pl.BlockSpec((1, _R, 64), lambda i: (1, i, 0)),
          pl.BlockSpec((1, 64), lambda i: (0, 0)),
      ],
      out_specs=pl.BlockSpec((_R, 64), lambda i: (i, 0)),
      out_shape=jax.ShapeDtypeStruct((N, 64), jnp.float32),
  )(q, q, b2)


def kernel(input, edge_index, adj_values, W1, b1, W2, b2):
  z128 = jnp.zeros((N_PAD, 128), jnp.float32)
  z64 = jnp.zeros((N_PAD, 64), jnp.float32)

  p = _spmm128(input, edge_index, adj_values, z128)
  z = _dense1(p, W1, b1.reshape(1, 128), W2)
  q = _spmm64(z, edge_index, adj_values, z64)
  return _dense2(q, b2.reshape(1, 64))


# speedup vs baseline: 1.2906x; 1.0033x over previous
import functools

import numpy as np

import jax
import jax.numpy as jnp
from jax import lax
from jax.experimental import pallas as pl
from jax.experimental.pallas import tpu as pltpu
from jax.experimental.pallas import tpu_sc as plsc

N = 10000
E = 320000
NC = 2
NS = 16
NW = NC * NS
EW = E // NW
C = 80
NCHUNK = EW // C
ROWS_PER_SUB = N // NS


NE = 8
NMAIN = 120
NTAIL = NCHUNK - NMAIN


def _make_spmm(d, nr, wgap, packed=False):
  mesh = plsc.VectorSubcoreMesh(core_axis_name="c", subcore_axis_name="s")

  @functools.partial(
      pl.kernel,
      out_type=jax.ShapeDtypeStruct((NC, N, d), jnp.float32),
      mesh=mesh,
      compiler_params=pltpu.CompilerParams(use_tc_tiling_on_sc=False,
                                           needs_layout_passes=False),
      scratch_types=(
          [pltpu.VMEM((2, C), jnp.int32) for _ in range(NE)]
          + [pltpu.VMEM((C,), jnp.float32) for _ in range(NE)]
          + [pltpu.VMEM((C, d), jnp.float32) for _ in range(nr)]
          + ([pltpu.VMEM((C, d // 2), jnp.int32) for _ in range(nr)]
             if packed else [])
          + [pltpu.VMEM_SHARED((N, d), jnp.float32)]
          + [pltpu.SemaphoreType.DMA for _ in range(2 * NE + nr)]
      ),
  )
  def spmm(x_hbm, ei_hbm, vals_hbm, zeros_hbm, out_hbm, *rest):
    ebuf = rest[:NE]
    valb = rest[NE:2 * NE]
    rows = rest[2 * NE:2 * NE + nr]
    ng = 2 * nr if packed else nr
    gbuf = rest[2 * NE + nr:2 * NE + ng] if packed else rows
    accum = rest[2 * NE + ng]
    esem = rest[2 * NE + ng + 1:2 * NE + ng + 1 + NE]
    ssem = rest[2 * NE + ng + 1 + NE:2 * NE + ng + 1 + 2 * NE]
    gsem = rest[2 * NE + ng + 1 + 2 * NE:]
    cid = lax.axis_index("c")
    sid = lax.axis_index("s")
    wid = cid * NS + sid
    ebase = wid * EW

    def load_idx(chunk, be):
      off = ebase + chunk * C
      pltpu.async_copy(ei_hbm.at[:, pl.ds(off, C)], ebuf[be], esem[be])
      pltpu.async_copy(vals_hbm.at[pl.ds(off, C)], valb[be], esem[be])

    def wait_idx(be):
      pltpu.make_async_copy(ei_hbm.at[:, pl.ds(0, C)], ebuf[be],
                            esem[be]).wait()
      pltpu.make_async_copy(vals_hbm.at[pl.ds(0, C)], valb[be],
                            esem[be]).wait()

    def gather(be, br):
      pltpu.async_copy(x_hbm.at[ebuf[be].at[1]], gbuf[br], gsem[br])

    def wait_gather(be, br):
      pltpu.make_async_copy(x_hbm.at[ebuf[be].at[1]], gbuf[br],
                            gsem[br]).wait()

    def scatter(be, br):
      pltpu.async_copy(rows[br], accum.at[ebuf[be].at[0]], ssem[be], add=True)

    def wait_scatter(be, br):
      pltpu.make_async_copy(rows[br], accum.at[ebuf[be].at[0]],
                            ssem[be]).wait()

    def scale(be, br):
      def group_body(g, c2):
        val16 = valb[be][pl.ds(g * 16, 16)]
        for l in range(16):
          e = g * 16 + l
          v = val16[l]
          if packed:
            for j in range(d // 32):
              v32 = gbuf[br][e, pl.ds(j * 16, 16)]
              lo = plsc.bitcast(v32 << 16, jnp.float32)
              hi = plsc.bitcast(v32 & jnp.int32(-65536), jnp.float32)
              rows[br][e, pl.ds(j * 32, 16)] = lo * v
              rows[br][e, pl.ds(j * 32 + 16, 16)] = hi * v
          else:
            for j in range(d // 16):
              sl = pl.ds(j * 16, 16)
              rows[br][e, sl] = rows[br][e, sl] * v
        return c2
      lax.fori_loop(0, C // 16, group_body, 0)

    for c in range(3):
      load_idx(c, c)
    for c in range(2):
      wait_idx(c)
      gather(c, c)

    r0 = sid * ROWS_PER_SUB
    pltpu.sync_copy(zeros_hbm.at[pl.ds(r0, ROWS_PER_SUB), :],
                    accum.at[pl.ds(r0, ROWS_PER_SUB), :])
    plsc.subcore_barrier()

    def step(k, k8, u):
      if u < wgap:
        @pl.when(k8 >= 1)
        def _wait_prev_scatter():
          wait_scatter((u - wgap) % NE, (u - wgap) % nr)
      else:
        wait_scatter((u - wgap) % NE, (u - wgap) % nr)
      load_idx(k + 3, (u + 3) % NE)
      wait_idx((u + 2) % NE)
      gather((u + 2) % NE, (u + 2) % nr)
      wait_gather(u % NE, u % nr)
      scale(u % NE, u % nr)
      scatter(u % NE, u % nr)

    def outer_body(k8, carry):
      for u in range(NE):
        step(k8 * NE + u, k8, u)
      return carry

    lax.fori_loop(0, NMAIN // NE, outer_body, 0)

    for k in range(NMAIN, NCHUNK):
      wait_scatter((k - wgap) % NE, (k - wgap) % nr)
      if k + 3 < NCHUNK:
        load_idx(k + 3, (k + 3) % NE)
      if k + 2 < NCHUNK:
        wait_idx((k + 2) % NE)
        gather((k + 2) % NE, (k + 2) % nr)
      wait_gather(k % NE, k % nr)
      scale(k % NE, k % nr)
      scatter(k % NE, k % nr)
    for k in range(NCHUNK - wgap, NCHUNK):
      wait_scatter(k % NE, k % nr)
    plsc.subcore_barrier()

    pltpu.sync_copy(accum.at[pl.ds(r0, ROWS_PER_SUB), :],
                    out_hbm.at[cid, pl.ds(r0, ROWS_PER_SUB), :])

  return spmm


_spmm128 = _make_spmm(128, nr=4, wgap=2)
_spmm64 = _make_spmm(64, nr=8, wgap=4)

_R = 1000


def _dense1_body(p0, p1, w1, b1, w2, out):
  y = p0[0] + p1[0]
  h = jnp.maximum(
      lax.dot_general(y, w1[...], (((1,), (0,)), ((), ())),
                      preferred_element_type=jnp.float32) + b1[...], 0.0)
  out[...] = lax.dot_general(h, w2[...], (((1,), (0,)), ((), ())),
                             preferred_element_type=jnp.float32)


def _dense1(p, w1, b1, w2):
  grid = (N // _R,)
  return pl.pallas_call(
      _dense1_body,
      grid=grid,
      in_specs=[
          pl.BlockSpec((1, _R, 128), lambda i: (0, i, 0)),
          pl.BlockSpec((1, _R, 128), lambda i: (1, i, 0)),
          pl.BlockSpec((128, 128), lambda i: (0, 0)),
          pl.BlockSpec((1, 128), lambda i: (0, 0)),
          pl.BlockSpec((128, 64), lambda i: (0, 0)),
      ],
      out_specs=pl.BlockSpec((_R, 64), lambda i: (i, 0)),
      out_shape=jax.ShapeDtypeStruct((N, 64), jnp.float32),
  )(p, p, w1, b1, w2)


def _dense2_body(q0, q1, b2, out):
  y = q0[0] + q1[0] + b2[...]
  m = jnp.max(y, axis=1, keepdims=True)
  s = y - m
  out[...] = s - jnp.log(jnp.sum(jnp.exp(s), axis=1, keepdims=True))


def _dense2(q, b2):
  grid = (N // _R,)
  return pl.pallas_call(
      _dense2_body,
      grid=grid,
      in_specs=[
          pl.BlockSpec((1, _R, 64), lambda i: (0, i, 0)),
          pl.BlockSpec((1, _R, 64), lambda i: (1, i, 0)),
          pl.BlockSpec((1, 64), lambda i: (0, 0)),
      ],
      out_specs=pl.BlockSpec((_R, 64), lambda i: (i, 0)),
      out_shape=jax.ShapeDtypeStruct((N, 64), jnp.float32),
  )(q, q, b2)


def kernel(input, edge_index, adj_values, W1, b1, W2, b2):
  z128 = jnp.zeros((N, 128), jnp.float32)
  z64 = jnp.zeros((N, 64), jnp.float32)

  p = _spmm128(input, edge_index, adj_values, z128)
  z = _dense1(p, W1, b1.reshape(1, 128), W2)
  q = _spmm64(z, edge_index, adj_values, z64)
  return _dense2(q, b2.reshape(1, 64))


# speedup vs baseline: 1.2954x; 1.0037x over previous
import functools

import jax
import jax.numpy as jnp
from jax import lax
from jax.experimental import pallas as pl
from jax.experimental.pallas import tpu as pltpu
from jax.experimental.pallas import tpu_sc as plsc

N = 10000
E = 320000
NC = 2
NS = 16
NW = NC * NS
EW = E // NW
C = 80
NCHUNK = EW // C
ROWS_PER_SUB = N // NS


NE = 8
NMAIN = 120
NTAIL = NCHUNK - NMAIN


def _make_spmm(d, nr, wgap, packed=False):
  mesh = plsc.VectorSubcoreMesh(core_axis_name="c", subcore_axis_name="s")

  @functools.partial(
      pl.kernel,
      out_type=jax.ShapeDtypeStruct((NC, N, d), jnp.float32),
      mesh=mesh,
      compiler_params=pltpu.CompilerParams(use_tc_tiling_on_sc=False,
                                           needs_layout_passes=False),
      scratch_types=(
          [pltpu.VMEM((2, C), jnp.int32) for _ in range(NE)]
          + [pltpu.VMEM((C,), jnp.float32) for _ in range(NE)]
          + [pltpu.VMEM((C, d), jnp.float32) for _ in range(nr)]
          + ([pltpu.VMEM((C, d // 2), jnp.int32) for _ in range(nr)]
             if packed else [])
          + [pltpu.VMEM_SHARED((N, d), jnp.float32)]
          + [pltpu.SemaphoreType.DMA for _ in range(2 * NE + nr)]
      ),
  )
  def spmm(x_hbm, ei_hbm, vals_hbm, zeros_hbm, out_hbm, *rest):
    ebuf = rest[:NE]
    valb = rest[NE:2 * NE]
    rows = rest[2 * NE:2 * NE + nr]
    ng = 2 * nr if packed else nr
    gbuf = rest[2 * NE + nr:2 * NE + ng] if packed else rows
    accum = rest[2 * NE + ng]
    esem = rest[2 * NE + ng + 1:2 * NE + ng + 1 + NE]
    ssem = rest[2 * NE + ng + 1 + NE:2 * NE + ng + 1 + 2 * NE]
    gsem = rest[2 * NE + ng + 1 + 2 * NE:]
    cid = lax.axis_index("c")
    sid = lax.axis_index("s")
    wid = cid * NS + sid
    ebase = wid * EW

    def load_idx(chunk, be):
      off = ebase + chunk * C
      pltpu.async_copy(ei_hbm.at[:, pl.ds(off, C)], ebuf[be], esem[be])
      pltpu.async_copy(vals_hbm.at[pl.ds(off, C)], valb[be], esem[be])

    def wait_idx(be):
      pltpu.make_async_copy(ei_hbm.at[:, pl.ds(0, C)], ebuf[be],
                            esem[be]).wait()
      pltpu.make_async_copy(vals_hbm.at[pl.ds(0, C)], valb[be],
                            esem[be]).wait()

    def gather(be, br):
      pltpu.async_copy(x_hbm.at[ebuf[be].at[1]], gbuf[br], gsem[br])

    def wait_gather(be, br):
      pltpu.make_async_copy(x_hbm.at[ebuf[be].at[1]], gbuf[br],
                            gsem[br]).wait()

    def scatter(be, br):
      pltpu.async_copy(rows[br], accum.at[ebuf[be].at[0]], ssem[be], add=True)

    def wait_scatter(be, br):
      pltpu.make_async_copy(rows[br], accum.at[ebuf[be].at[0]],
                            ssem[be]).wait()

    def scale(be, br):
      def group_body(g, c2):
        val16 = valb[be][pl.ds(g * 16, 16)]
        for l in range(16):
          e = g * 16 + l
          v = val16[l]
          if packed:
            for j in range(d // 32):
              v32 = gbuf[br][e, pl.ds(j * 16, 16)]
              lo = plsc.bitcast(v32 << 16, jnp.float32)
              hi = plsc.bitcast(v32 & jnp.int32(-65536), jnp.float32)
              rows[br][e, pl.ds(j * 32, 16)] = lo * v
              rows[br][e, pl.ds(j * 32 + 16, 16)] = hi * v
          else:
            for j in range(d // 16):
              sl = pl.ds(j * 16, 16)
              rows[br][e, sl] = rows[br][e, sl] * v
        return c2
      lax.fori_loop(0, C // 16, group_body, 0)

    for c in range(3):
      load_idx(c, c)
    for c in range(2):
      wait_idx(c)
      gather(c, c)

    r0 = sid * ROWS_PER_SUB
    pltpu.sync_copy(zeros_hbm.at[pl.ds(r0, ROWS_PER_SUB), :],
                    accum.at[pl.ds(r0, ROWS_PER_SUB), :])
    plsc.subcore_barrier()

    def step(k, k8, u):
      if u < wgap:
        @pl.when(k8 >= 1)
        def _wait_prev_scatter():
          wait_scatter((u - wgap) % NE, (u - wgap) % nr)
      else:
        wait_scatter((u - wgap) % NE, (u - wgap) % nr)
      load_idx(k + 3, (u + 3) % NE)
      wait_idx((u + 2) % NE)
      gather((u + 2) % NE, (u + 2) % nr)
      wait_gather(u % NE, u % nr)
      scale(u % NE, u % nr)
      scatter(u % NE, u % nr)

    def outer_body(k8, carry):
      for u in range(NE):
        step(k8 * NE + u, k8, u)
      return carry

    lax.fori_loop(0, NMAIN // NE, outer_body, 0)

    for k in range(NMAIN, NCHUNK):
      wait_scatter((k - wgap) % NE, (k - wgap) % nr)
      if k + 3 < NCHUNK:
        load_idx(k + 3, (k + 3) % NE)
      if k + 2 < NCHUNK:
        wait_idx((k + 2) % NE)
        gather((k + 2) % NE, (k + 2) % nr)
      wait_gather(k % NE, k % nr)
      scale(k % NE, k % nr)
      scatter(k % NE, k % nr)
    for k in range(NCHUNK - wgap, NCHUNK):
      wait_scatter(k % NE, k % nr)
    plsc.subcore_barrier()

    pltpu.sync_copy(accum.at[pl.ds(r0, ROWS_PER_SUB), :],
                    out_hbm.at[cid, pl.ds(r0, ROWS_PER_SUB), :])

  return spmm


_spmm128 = _make_spmm(128, nr=4, wgap=2)
_spmm64 = _make_spmm(64, nr=8, wgap=4)

_R = 1000


def _dense1_body(p0, p1, w1, b1, w2, out):
  y = p0[0] + p1[0]
  h = jnp.maximum(
      lax.dot_general(y, w1[...], (((1,), (0,)), ((), ())),
                      preferred_element_type=jnp.float32) + b1[...], 0.0)
  out[...] = lax.dot_general(h, w2[...], (((1,), (0,)), ((), ())),
                             preferred_element_type=jnp.float32)


def _dense1(p, w1, b1, w2):
  grid = (N // _R,)
  return pl.pallas_call(
      _dense1_body,
      grid=grid,
      in_specs=[
          pl.BlockSpec((1, _R, 128), lambda i: (0, i, 0)),
          pl.BlockSpec((1, _R, 128), lambda i: (1, i, 0)),
          pl.BlockSpec((128, 128), lambda i: (0, 0)),
          pl.BlockSpec((1, 128), lambda i: (0, 0)),
          pl.BlockSpec((128, 64), lambda i: (0, 0)),
      ],
      out_specs=pl.BlockSpec((_R, 64), lambda i: (i, 0)),
      out_shape=jax.ShapeDtypeStruct((N, 64), jnp.float32),
  )(p, p, w1, b1, w2)


def _dense2_body(q0, q1, b2, out):
  y = q0[0] + q1[0] + b2[...]
  m = jnp.max(y, axis=1, keepdims=True)
  s = y - m
  out[...] = s - jnp.log(jnp.sum(jnp.exp(s), axis=1, keepdims=True))


def _dense2(q, b2):
  grid = (N // _R,)
  return pl.pallas_call(
      _dense2_body,
      grid=grid,
      in_specs=[
          pl.BlockSpec((1, _R, 64), lambda i: (0, i, 0)),
          pl.BlockSpec((1, _R, 64), lambda i: (1, i, 0)),
          pl.BlockSpec((1, 64), lambda i: (0, 0)),
      ],
      out_specs=pl.BlockSpec((_R, 64), lambda i: (i, 0)),
      out_shape=jax.ShapeDtypeStruct((N, 64), jnp.float32),
  )(q, q, b2)


def kernel(input, edge_index, adj_values, W1, b1, W2, b2):
  z128 = jnp.zeros((N, 128), jnp.float32)
  z64 = jnp.zeros((N, 64), jnp.float32)

  p = _spmm128(input, edge_index, adj_values, z128)
  z = _dense1(p, W1, b1.reshape(1, 128), W2)
  q = _spmm64(z, edge_index, adj_values, z64)
  return _dense2(q, b2.reshape(1, 64))
